# baseline (speedup 1.0000x reference)
import jax
import jax.numpy as jnp
from jax.experimental import pallas as pl
from jax.experimental.pallas import tpu as pltpu

_A, _B, _SD, _AD, _H = 8, 4096, 112, 16, 128
_SPARSE = 0.05
_TB = 512
_NT = _B // _TB
_TS = 1024
_NS = _B // _TS
_F32 = jnp.float32


def _leaky(x):
    return jnp.maximum(x, 0.01 * x)


def _dot(a, b):
    return jnp.dot(a, b, preferred_element_type=_F32)


def _prep(st_ref, ac_ref, trs_ref,
          k_sa_Ws_ref, k_sa_Wa_ref,
          k_trW1_ref, k_trb1_ref, k_trW2_ref,
          k_encB_ref, k_enc_b_ref, k_dec_W1_ref,
          l_sa_Ws_ref, l_sa_Wa_ref,
          l_trW1_ref, l_trb1_ref, l_trW2_ref,
          l_encB_ref, l_enc_b1_ref, gcn_W_ref, gcn_b_ref,
          ms_ref, ss_ref, ma_ref, sa_ref,
          kb2_ref, lb2_ref, gw1_ref, gb1_ref,
          sums_sc, sqs_sc, suma_sc, sqa_sc):
    t = pl.program_id(0)

    xs = st_ref[...]
    xa = ac_ref[...]
    ssum = jnp.sum(xs, axis=1)
    ssq = jnp.sum(xs * xs, axis=1)
    asum = jnp.sum(xa, axis=1)
    asq = jnp.sum(xa * xa, axis=1)

    @pl.when(t == 0)
    def _():
        sums_sc[...] = ssum
        sqs_sc[...] = ssq
        suma_sc[...] = asum
        sqa_sc[...] = asq

    @pl.when(t > 0)
    def _():
        sums_sc[...] = sums_sc[...] + ssum
        sqs_sc[...] = sqs_sc[...] + ssq
        suma_sc[...] = suma_sc[...] + asum
        sqa_sc[...] = sqa_sc[...] + asq

    @pl.when(t == _NS - 1)
    def _finalize():
        ms = sums_sc[...] * (1.0 / _B)
        vs = sqs_sc[...] * (1.0 / _B) - ms * ms
        ms_ref[...] = ms
        ss_ref[...] = jax.lax.rsqrt(vs + 1e-5)
        ma = suma_sc[...] * (1.0 / _B)
        va = sqa_sc[...] * (1.0 / _B) - ma * ma
        ma_ref[...] = ma
        sa_ref[...] = jax.lax.rsqrt(va + 1e-5)
        trs_col = trs_ref[...]
        tvec = _leaky(trs_col * k_trW1_ref[...] + k_trb1_ref[...])
        t2vec = _leaky(trs_col * l_trW1_ref[...] + l_trb1_ref[...])
        for a in range(_A):
            ktr = _leaky(_dot(tvec[a:a + 1, :], k_trW2_ref[a]))
            kb2_ref[a:a + 1, :] = (_dot(ktr, k_encB_ref[a])
                                   + k_enc_b_ref[a:a + 1, :])
            ltr = _leaky(_dot(t2vec[a:a + 1, :], l_trW2_ref[a]))
            lb2_ref[a:a + 1, :] = (_dot(ltr, l_encB_ref[a])
                                   + l_enc_b1_ref[a:a + 1, :])
            gw1_ref[a, :, :] = _dot(gcn_W_ref[...], k_dec_W1_ref[a])
            gb1_ref[a:a + 1, :] = _dot(gcn_b_ref[...], k_dec_W1_ref[a])


def _main(st_ref, ac_ref, cc_ref,
          k_sa_Ws_ref, k_sa_Wa_ref, k_sa_b_ref, k_encA_ref,
          k_dec_b1_ref, k_dec_W2_ref,
          l_sa_Ws_ref, l_sa_Wa_ref, l_sa_b_ref, l_encA_ref, l_enc_W2_ref,
          ms_ref, ss_ref, ma_ref, sa_ref,
          kb2_ref, lb2_ref, gw1_ref, gb1_ref,
          out_ref, kenc_sc):
    xs_all = st_ref[...]
    xa_all = ac_ref[...]
    lams = []
    for a in range(_A):
        xs = (xs_all[a] - ms_ref[a:a + 1, :]) * ss_ref[a:a + 1, :]
        xa = (xa_all[a] - ma_ref[a:a + 1, :]) * sa_ref[a:a + 1, :]
        ksa = _leaky(_dot(xs, k_sa_Ws_ref[a]) + _dot(xa, k_sa_Wa_ref[a])
                     + k_sa_b_ref[a:a + 1, :])
        kenc_sc[a, :, :] = _leaky(_dot(ksa, k_encA_ref[a])
                                  + kb2_ref[a:a + 1, :])
        lsa = _leaky(_dot(xs, l_sa_Ws_ref[a]) + _dot(xa, l_sa_Wa_ref[a])
                     + l_sa_b_ref[a:a + 1, :])
        e1 = _leaky(_dot(lsa, l_encA_ref[a]) + lb2_ref[a:a + 1, :])
        lams.append(_leaky(_dot(e1, l_enc_W2_ref[a])))

    cc = cc_ref[...]
    lane = jax.lax.broadcasted_iota(jnp.int32, (_TB, _A * _A), 1)
    isdiag = (lane % (_A + 1)) == 0
    mask = jnp.where((cc >= _SPARSE) | isdiag, 1.0, 0.0)
    w = mask * cc
    deg = mask[:, 0:_A]
    for i in range(1, _A):
        deg = deg + mask[:, i * _A:(i + 1) * _A]
    dis = jax.lax.rsqrt(deg)
    wn_parts = [w[:, i * _A:(i + 1) * _A] * dis * dis[:, i:i + 1]
                for i in range(_A)]
    rs = wn_parts[0]
    for i in range(1, _A):
        rs = rs + wn_parts[i]
    cols = []
    for j in range(_A):
        agg = wn_parts[0][:, j:j + 1] * kenc_sc[0]
        for i in range(1, _A):
            agg = agg + wn_parts[i][:, j:j + 1] * kenc_sc[i]
        d1 = _leaky(_dot(agg, gw1_ref[j])
                    + rs[:, j:j + 1] * gb1_ref[j:j + 1, :]
                    + k_dec_b1_ref[j:j + 1, :])
        kk = _leaky(_dot(d1, k_dec_W2_ref[j]))
        cols.append(lams[j] * kk)
    out_ref[...] = jnp.concatenate(cols, axis=1)


def kernel(states, actions, trs, ccs, k_sa_W, k_sa_b, k_tr_W1, k_tr_b1,
           k_tr_W2, k_enc_W, k_enc_b, k_dec_W1, k_dec_b1, k_dec_W2,
           l_sa_W, l_sa_b, l_tr_W1, l_tr_b1, l_tr_W2, l_enc_W1, l_enc_b1,
           l_enc_W2, gcn_W, gcn_b):
    cc2 = ccs.reshape(_B, _A * _A)
    trs_col = trs.reshape(_A, 1)
    k_trW1 = k_tr_W1.reshape(_A, _H)
    l_trW1 = l_tr_W1.reshape(_A, _H)
    gcn_b2 = gcn_b.reshape(1, _H)
    k_sa_Ws = k_sa_W[:, :_SD, :]
    k_sa_Wa = k_sa_W[:, _SD:, :]
    l_sa_Ws = l_sa_W[:, :_SD, :]
    l_sa_Wa = l_sa_W[:, _SD:, :]
    k_encA = k_enc_W[:, :_H, :]
    k_encB = k_enc_W[:, _H:, :]
    l_encA = l_enc_W1[:, :_H, :]
    l_encB = l_enc_W1[:, _H:, :]

    def fixed(ndim):
        return lambda *_: (0,) * ndim

    prep_in_specs = [
        pl.BlockSpec((_A, _TS, _SD), lambda t: (0, t, 0)),
        pl.BlockSpec((_A, _TS, _AD), lambda t: (0, t, 0)),
        pl.BlockSpec((_A, 1), fixed(2)),
        pl.BlockSpec((_A, _SD, _H), fixed(3)),
        pl.BlockSpec((_A, _AD, _H), fixed(3)),
        pl.BlockSpec((_A, _H), fixed(2)),
        pl.BlockSpec((_A, _H), fixed(2)),
        pl.BlockSpec((_A, _H, _H), fixed(3)),
        pl.BlockSpec((_A, _H, _H), fixed(3)),
        pl.BlockSpec((_A, _H), fixed(2)),
        pl.BlockSpec((_A, _H, _H), fixed(3)),
        pl.BlockSpec((_A, _SD, _H), fixed(3)),
        pl.BlockSpec((_A, _AD, _H), fixed(3)),
        pl.BlockSpec((_A, _H), fixed(2)),
        pl.BlockSpec((_A, _H), fixed(2)),
        pl.BlockSpec((_A, _H, _H), fixed(3)),
        pl.BlockSpec((_A, _H, _H), fixed(3)),
        pl.BlockSpec((_A, _H), fixed(2)),
        pl.BlockSpec((_H, _H), fixed(2)),
        pl.BlockSpec((1, _H), fixed(2)),
    ]
    prep_out_specs = (
        pl.BlockSpec((_A, _SD), fixed(2)),
        pl.BlockSpec((_A, _SD), fixed(2)),
        pl.BlockSpec((_A, _AD), fixed(2)),
        pl.BlockSpec((_A, _AD), fixed(2)),
        pl.BlockSpec((_A, _H), fixed(2)),
        pl.BlockSpec((_A, _H), fixed(2)),
        pl.BlockSpec((_A, _H, _H), fixed(3)),
        pl.BlockSpec((_A, _H), fixed(2)),
    )
    prep_out_shapes = (
        jax.ShapeDtypeStruct((_A, _SD), _F32),
        jax.ShapeDtypeStruct((_A, _SD), _F32),
        jax.ShapeDtypeStruct((_A, _AD), _F32),
        jax.ShapeDtypeStruct((_A, _AD), _F32),
        jax.ShapeDtypeStruct((_A, _H), _F32),
        jax.ShapeDtypeStruct((_A, _H), _F32),
        jax.ShapeDtypeStruct((_A, _H, _H), _F32),
        jax.ShapeDtypeStruct((_A, _H), _F32),
    )

    ms, ss, ma, sa, kb2, lb2, gw1, gb1 = pl.pallas_call(
        _prep,
        grid=(_NS,),
        in_specs=prep_in_specs,
        out_specs=prep_out_specs,
        out_shape=prep_out_shapes,
        scratch_shapes=[
            pltpu.VMEM((_A, _SD), _F32),
            pltpu.VMEM((_A, _SD), _F32),
            pltpu.VMEM((_A, _AD), _F32),
            pltpu.VMEM((_A, _AD), _F32),
        ],
    )(states, actions, trs_col,
      k_sa_Ws, k_sa_Wa, k_trW1, k_tr_b1, k_tr_W2,
      k_encB, k_enc_b, k_dec_W1,
      l_sa_Ws, l_sa_Wa, l_trW1, l_tr_b1, l_tr_W2,
      l_encB, l_enc_b1, gcn_W, gcn_b2)

    main_in_specs = [
        pl.BlockSpec((_A, _TB, _SD), lambda t: (0, t, 0)),
        pl.BlockSpec((_A, _TB, _AD), lambda t: (0, t, 0)),
        pl.BlockSpec((_TB, _A * _A), lambda t: (t, 0)),
        pl.BlockSpec((_A, _SD, _H), fixed(3)),
        pl.BlockSpec((_A, _AD, _H), fixed(3)),
        pl.BlockSpec((_A, _H), fixed(2)),
        pl.BlockSpec((_A, _H, _H), fixed(3)),
        pl.BlockSpec((_A, _H), fixed(2)),
        pl.BlockSpec((_A, _H, 1), fixed(3)),
        pl.BlockSpec((_A, _SD, _H), fixed(3)),
        pl.BlockSpec((_A, _AD, _H), fixed(3)),
        pl.BlockSpec((_A, _H), fixed(2)),
        pl.BlockSpec((_A, _H, _H), fixed(3)),
        pl.BlockSpec((_A, _H, 1), fixed(3)),
        pl.BlockSpec((_A, _SD), fixed(2)),
        pl.BlockSpec((_A, _SD), fixed(2)),
        pl.BlockSpec((_A, _AD), fixed(2)),
        pl.BlockSpec((_A, _AD), fixed(2)),
        pl.BlockSpec((_A, _H), fixed(2)),
        pl.BlockSpec((_A, _H), fixed(2)),
        pl.BlockSpec((_A, _H, _H), fixed(3)),
        pl.BlockSpec((_A, _H), fixed(2)),
    ]

    out = pl.pallas_call(
        _main,
        grid=(_NT,),
        in_specs=main_in_specs,
        out_specs=pl.BlockSpec((_TB, _A), lambda t: (t, 0)),
        out_shape=jax.ShapeDtypeStruct((_B, _A), _F32),
        scratch_shapes=[
            pltpu.VMEM((_A, _TB, _H), _F32),
        ],
        compiler_params=pltpu.CompilerParams(
            dimension_semantics=("parallel",),
        ),
    )(states, actions, cc2,
      k_sa_Ws, k_sa_Wa, k_sa_b, k_encA, k_dec_b1, k_dec_W2,
      l_sa_Ws, l_sa_Wa, l_sa_b, l_encA, l_enc_W2,
      ms, ss, ma, sa, kb2, lb2, gw1, gb1)
    return out
